# Initial kernel scaffold; baseline (speedup 1.0000x reference)
import jax
import jax.numpy as jnp
from jax.experimental import pallas as pl


def kernel(features, labels, conf_weights, prototypes):
    raise NotImplementedError("write your pallas kernel here")


# speedup vs baseline: 7.9078x; 7.9078x over previous
import functools

import jax
import jax.numpy as jnp
from jax import lax
from jax.experimental import pallas as pl
from jax.experimental.pallas import tpu as pltpu
from jax.experimental.pallas import tpu_sc as plsc

_C = 100000
_D = 128
_N = 1024
_MOM = 0.9
_TEMP = 0.15
_INV_TEMP = 1.0 / _TEMP

_CB = 2000
_NBLK = _C // _CB


_SC_NC = 2
_SC_NS = 16
_NW = _SC_NC * _SC_NS
_BPW = _N // _NW


@functools.lru_cache(maxsize=1)
def _sc_gather_fn():
    @functools.partial(
        pl.kernel,
        mesh=plsc.VectorSubcoreMesh(core_axis_name="c", subcore_axis_name="s"),
        out_type=jax.ShapeDtypeStruct((_N, _D), jnp.float32),
        scratch_types=[
            pltpu.VMEM((_BPW,), jnp.int32),
            pltpu.VMEM((_BPW, _D), jnp.float32),
            pltpu.SemaphoreType.DMA,
        ],
    )
    def _sc_gather(table_hbm, idx_hbm, out_hbm, idx_v, rows_v, sem):
        wid = lax.axis_index("s") * _SC_NC + lax.axis_index("c")
        base = wid * _BPW
        pltpu.sync_copy(idx_hbm.at[pl.ds(base, _BPW)], idx_v)
        pltpu.async_copy(table_hbm.at[idx_v], rows_v, sem).wait()
        pltpu.sync_copy(rows_v, out_hbm.at[pl.ds(base, _BPW)])

    return _sc_gather


def _row_normalize(x):
    n = jnp.sqrt(jnp.sum(x * x, axis=1, keepdims=True))
    return x / jnp.clip(n, 1e-12)


def _tc_body(feat_ref, lcol_ref, lrow_ref, cw_ref, pg_ref, protos_ref,
             out_ref, f_s, acc_s):
    i = pl.program_id(0)

    @pl.when(i == 0)
    def _init():
        f_s[...] = _row_normalize(feat_ref[...])
        acc_s[...] = jnp.zeros_like(acc_s)

    f = f_s[...]
    blk = protos_ref[...]
    logits = lax.dot_general(
        f, blk, (((1,), (1,)), ((), ())),
        preferred_element_type=jnp.float32) * _INV_TEMP
    acc_s[...] += jnp.sum(jnp.exp(logits), axis=1, keepdims=True)

    @pl.when(i == _NBLK - 1)
    def _final():
        lcol = lcol_ref[...]
        lrow = lrow_ref[...]
        S = (lcol == lrow).astype(jnp.float32)
        counts = jnp.sum(S, axis=1, keepdims=True)
        sums = jnp.dot(S, f, preferred_element_type=jnp.float32)
        mean = _row_normalize(sums / jnp.clip(counts, 1.0))
        pg = pg_ref[...]
        upd = _row_normalize(_MOM * pg + (1.0 - _MOM) * mean)
        old_l = lax.dot_general(
            f, pg, (((1,), (1,)), ((), ())),
            preferred_element_type=jnp.float32) * _INV_TEMP
        new_l = lax.dot_general(
            f, upd, (((1,), (1,)), ((), ())),
            preferred_element_type=jnp.float32) * _INV_TEMP
        w_row = 1.0 / jnp.sum(S, axis=0, keepdims=True)
        delta = jnp.sum((jnp.exp(new_l) - jnp.exp(old_l)) * w_row,
                        axis=1, keepdims=True)
        z = acc_s[...] + delta
        ri = lax.broadcasted_iota(jnp.int32, (_N, _N), 0)
        ci = lax.broadcasted_iota(jnp.int32, (_N, _N), 1)
        diag = jnp.sum(jnp.where(ri == ci, new_l, 0.0), axis=1,
                       keepdims=True)
        cw = cw_ref[...]
        nll = jnp.log(z) - diag
        num = jnp.sum(nll * cw, axis=(0, 1), keepdims=True)
        den = jnp.sum(cw, axis=(0, 1), keepdims=True)
        out_ref[...] = num / jnp.clip(den, 1e-12)


def _tc_loss(features, labels_col, labels_row, conf_w, pg, prototypes):
    return pl.pallas_call(
        _tc_body,
        grid=(_NBLK,),
        in_specs=[
            pl.BlockSpec((_N, _D), lambda i: (0, 0)),
            pl.BlockSpec((_N, 1), lambda i: (0, 0)),
            pl.BlockSpec((1, _N), lambda i: (0, 0)),
            pl.BlockSpec((_N, 1), lambda i: (0, 0)),
            pl.BlockSpec((_N, _D), lambda i: (0, 0)),
            pl.BlockSpec((_CB, _D), lambda i: (i, 0)),
        ],
        out_specs=pl.BlockSpec((1, 1), lambda i: (0, 0)),
        out_shape=jax.ShapeDtypeStruct((1, 1), jnp.float32),
        scratch_shapes=[
            pltpu.VMEM((_N, _D), jnp.float32),
            pltpu.VMEM((_N, 1), jnp.float32),
        ],
    )(features, labels_col, labels_row, conf_w, pg, prototypes)


def kernel(features, labels, conf_weights, prototypes):
    labels = labels.astype(jnp.int32)
    pg = _sc_gather_fn()(prototypes, labels)
    out = _tc_loss(
        features,
        labels.reshape(_N, 1),
        labels.reshape(1, _N),
        conf_weights.reshape(_N, 1),
        pg,
        prototypes,
    )
    return out[0, 0]
